# baseline (speedup 1.0000x reference)
import functools

import jax
import jax.numpy as jnp
from jax import lax
from jax.experimental import pallas as pl
from jax.experimental.pallas import tpu as pltpu
from jax.experimental.pallas import tpu_sc as plsc

N_GENE = 10000
N_DISEASE = 10000
E = 320000
D = 128
DW = D // 2

NC = 2
NS = 16
NW = NC * NS
PW = E // NW
G = 80
NCHUNK = PW // G


def _normalize_body(g_ref, d_ref, go_ref, do_ref):
    x = g_ref[...]
    go_ref[...] = (x / jnp.sqrt(jnp.sum(x * x, axis=1, keepdims=True))
                   ).astype(jnp.bfloat16)
    y = d_ref[...]
    do_ref[...] = (y / jnp.sqrt(jnp.sum(y * y, axis=1, keepdims=True))
                   ).astype(jnp.bfloat16)


def _normalize(h_gene, h_disease):
    return pl.pallas_call(
        _normalize_body,
        out_shape=(
            jax.ShapeDtypeStruct((N_GENE, D), jnp.bfloat16),
            jax.ShapeDtypeStruct((N_DISEASE, D), jnp.bfloat16),
        ),
    )(h_gene, h_disease)


@functools.partial(
    pl.kernel,
    mesh=plsc.VectorSubcoreMesh(core_axis_name="c", subcore_axis_name="s",
                                num_cores=NC),
    out_type=jax.ShapeDtypeStruct((E,), jnp.float32),
    compiler_params=pltpu.CompilerParams(needs_layout_passes=False,
                                         use_tc_tiling_on_sc=False),
    scratch_types=[
        pltpu.VMEM((PW,), jnp.int32),
        pltpu.VMEM((PW,), jnp.int32),
        pltpu.VMEM((G, DW), jnp.int32),
        pltpu.VMEM((G, DW), jnp.int32),
        pltpu.VMEM((G, DW), jnp.int32),
        pltpu.VMEM((G, DW), jnp.int32),
        pltpu.VMEM((PW,), jnp.float32),
        pltpu.SemaphoreType.DMA,
        pltpu.SemaphoreType.DMA,
    ],
)
def _edge_dots(hg_hbm, hd_hbm, src_hbm, dst_hbm, out_hbm,
               src_v, dst_v, u0, v0, u1, v1, o_v, s0, s1):
    wid = lax.axis_index("s") * NC + lax.axis_index("c")
    base = pl.multiple_of(wid * PW, 8)

    pltpu.sync_copy(src_hbm.at[pl.ds(base, PW)], src_v)
    pltpu.sync_copy(dst_hbm.at[pl.ds(base, PW)], dst_v)

    iota16 = lax.iota(jnp.int32, 16)

    def issue(c, ub, vb, sem):
        off = pl.multiple_of(c * G, 8)
        pltpu.async_copy(hg_hbm.at[src_v.at[pl.ds(off, G)]], ub, sem)
        pltpu.async_copy(hd_hbm.at[dst_v.at[pl.ds(off, G)]], vb, sem)

    def wait2(ub, vb, sem):
        pltpu.make_async_copy(hg_hbm.at[src_v.at[pl.ds(0, G)]], ub, sem).wait()
        pltpu.make_async_copy(hd_hbm.at[dst_v.at[pl.ds(0, G)]], vb, sem).wait()

    def dot16(ub, vb, rows, i):
        col = (iota16 + i) & (DW - 1)
        uw = plsc.load_gather(ub, [rows, col])
        vw = plsc.load_gather(vb, [rows, col])
        pw = plsc.bitcast(uw, jnp.bfloat16) * plsc.bitcast(vw, jnp.bfloat16)
        return plsc.unpack(pw, format=plsc.PackFormat.INTERLEAVED,
                           preferred_element_type=jnp.float32)

    def compute(c, ub, vb):
        off = pl.multiple_of(c * G, 8)
        zero = jnp.zeros((16,), jnp.float32)
        for g in range(G // 16):
            rows = iota16 + (g * 16)

            def w_body(i, accs):
                a0, a1, a2, a3 = accs
                p0, p1 = dot16(ub, vb, rows, 2 * i)
                q0, q1 = dot16(ub, vb, rows, 2 * i + 1)
                return (a0 + p0, a1 + p1, a2 + q0, a3 + q1)

            a0, a1, a2, a3 = lax.fori_loop(0, DW // 2, w_body,
                                           (zero, zero, zero, zero), unroll=4)
            o_v[pl.ds(off + g * 16, 16)] = (a0 + a1) + (a2 + a3)

    issue(0, u0, v0, s0)

    def pair_body(t, _):
        c0 = 2 * t
        issue(c0 + 1, u1, v1, s1)
        wait2(u0, v0, s0)
        compute(c0, u0, v0)
        issue(c0 + 2, u0, v0, s0)
        wait2(u1, v1, s1)
        compute(c0 + 1, u1, v1)
        return 0

    lax.fori_loop(0, (NCHUNK - 1) // 2, pair_body, 0)
    wait2(u0, v0, s0)
    compute(NCHUNK - 1, u0, v0)
    pltpu.sync_copy(o_v, out_hbm.at[pl.ds(base, PW)])


def _pack(x_bf16):
    n = x_bf16.shape[0]
    return jax.lax.bitcast_convert_type(
        x_bf16.reshape(n, DW, 2), jnp.int32)


def kernel(h_gene, h_disease, edge_index):
    gn, dn = _normalize(h_gene, h_disease)
    src = edge_index[0].astype(jnp.int32)
    dst = edge_index[1].astype(jnp.int32)
    out = _edge_dots(_pack(gn), _pack(dn), src, dst)
    return out.reshape(E, 1)


# speedup vs baseline: 1.0269x; 1.0269x over previous
import functools

import jax
import jax.numpy as jnp
from jax import lax
from jax.experimental import pallas as pl
from jax.experimental.pallas import tpu as pltpu
from jax.experimental.pallas import tpu_sc as plsc

N_GENE = 10000
N_DISEASE = 10000
E = 320000
D = 128
DW = D // 2

NC = 2
NS = 16
NW = NC * NS
PW = E // NW
G = 80
NCHUNK = PW // G


def _normalize_body(g_ref, d_ref, go_ref, do_ref):
    x = g_ref[...]
    go_ref[...] = (x / jnp.sqrt(jnp.sum(x * x, axis=1, keepdims=True))
                   ).astype(jnp.bfloat16)
    y = d_ref[...]
    do_ref[...] = (y / jnp.sqrt(jnp.sum(y * y, axis=1, keepdims=True))
                   ).astype(jnp.bfloat16)


def _normalize(h_gene, h_disease):
    return pl.pallas_call(
        _normalize_body,
        out_shape=(
            jax.ShapeDtypeStruct((N_GENE, D), jnp.bfloat16),
            jax.ShapeDtypeStruct((N_DISEASE, D), jnp.bfloat16),
        ),
    )(h_gene, h_disease)


@functools.partial(
    pl.kernel,
    mesh=plsc.VectorSubcoreMesh(core_axis_name="c", subcore_axis_name="s",
                                num_cores=NC),
    out_type=jax.ShapeDtypeStruct((E,), jnp.float32),
    compiler_params=pltpu.CompilerParams(needs_layout_passes=False,
                                         use_tc_tiling_on_sc=False),
    scratch_types=[
        pltpu.VMEM((PW,), jnp.int32),
        pltpu.VMEM((PW,), jnp.int32),
        pltpu.VMEM((G, DW), jnp.int32),
        pltpu.VMEM((G, DW), jnp.int32),
        pltpu.VMEM((G, DW), jnp.int32),
        pltpu.VMEM((G, DW), jnp.int32),
        pltpu.VMEM((G, DW), jnp.int32),
        pltpu.VMEM((G, DW), jnp.int32),
        pltpu.VMEM((G, DW), jnp.int32),
        pltpu.VMEM((G, DW), jnp.int32),
        pltpu.VMEM((PW,), jnp.float32),
        pltpu.SemaphoreType.DMA,
        pltpu.SemaphoreType.DMA,
        pltpu.SemaphoreType.DMA,
        pltpu.SemaphoreType.DMA,
    ],
)
def _edge_dots(hg_hbm, hd_hbm, src_hbm, dst_hbm, out_hbm,
               src_v, dst_v, u0, v0, u1, v1, u2, v2, u3, v3, o_v,
               s0, s1, s2, s3):
    wid = lax.axis_index("s") * NC + lax.axis_index("c")
    base = pl.multiple_of(wid * PW, 8)

    pltpu.sync_copy(src_hbm.at[pl.ds(base, PW)], src_v)
    pltpu.sync_copy(dst_hbm.at[pl.ds(base, PW)], dst_v)

    iota16 = lax.iota(jnp.int32, 16)

    def issue(c, ub, vb, sem):
        off = pl.multiple_of(c * G, 8)
        pltpu.async_copy(hg_hbm.at[src_v.at[pl.ds(off, G)]], ub, sem)
        pltpu.async_copy(hd_hbm.at[dst_v.at[pl.ds(off, G)]], vb, sem)

    def wait2(ub, vb, sem):
        pltpu.make_async_copy(hg_hbm.at[src_v.at[pl.ds(0, G)]], ub, sem).wait()
        pltpu.make_async_copy(hd_hbm.at[dst_v.at[pl.ds(0, G)]], vb, sem).wait()

    def dot16(ub, vb, rows, i):
        col = (iota16 + i) & (DW - 1)
        uw = plsc.load_gather(ub, [rows, col])
        vw = plsc.load_gather(vb, [rows, col])
        pw = plsc.bitcast(uw, jnp.bfloat16) * plsc.bitcast(vw, jnp.bfloat16)
        return plsc.unpack(pw, format=plsc.PackFormat.INTERLEAVED,
                           preferred_element_type=jnp.float32)

    def compute(c, ub, vb):
        off = pl.multiple_of(c * G, 8)
        zero = jnp.zeros((16,), jnp.float32)
        for g in range(G // 16):
            rows = iota16 + (g * 16)

            def w_body(i, accs):
                a0, a1, a2, a3 = accs
                p0, p1 = dot16(ub, vb, rows, 2 * i)
                q0, q1 = dot16(ub, vb, rows, 2 * i + 1)
                return (a0 + p0, a1 + p1, a2 + q0, a3 + q1)

            a0, a1, a2, a3 = lax.fori_loop(0, DW // 2, w_body,
                                           (zero, zero, zero, zero), unroll=4)
            o_v[pl.ds(off + g * 16, 16)] = (a0 + a1) + (a2 + a3)

    bufs = ((u0, v0, s0), (u1, v1, s1), (u2, v2, s2), (u3, v3, s3))
    for b in range(4):
        issue(b, *bufs[b])

    def quad_body(t, _):
        c0 = 4 * t
        for b in range(4):
            c = c0 + b
            ub, vb, sb = bufs[b]
            wait2(ub, vb, sb)
            compute(c, ub, vb)

            @pl.when(c + 4 < NCHUNK)
            def _():
                issue(c + 4, ub, vb, sb)
        return 0

    lax.fori_loop(0, (NCHUNK - 1) // 4, quad_body, 0)
    wait2(u0, v0, s0)
    compute(NCHUNK - 1, u0, v0)
    pltpu.sync_copy(o_v, out_hbm.at[pl.ds(base, PW)])


def _pack(x_bf16):
    n = x_bf16.shape[0]
    return jax.lax.bitcast_convert_type(
        x_bf16.reshape(n, DW, 2), jnp.int32)


def kernel(h_gene, h_disease, edge_index):
    gn, dn = _normalize(h_gene, h_disease)
    src = edge_index[0].astype(jnp.int32)
    dst = edge_index[1].astype(jnp.int32)
    out = _edge_dots(_pack(gn), _pack(dn), src, dst)
    return out.reshape(E, 1)


# speedup vs baseline: 1.2326x; 1.2002x over previous
import functools

import jax
import jax.numpy as jnp
from jax import lax
from jax.experimental import pallas as pl
from jax.experimental.pallas import tpu as pltpu
from jax.experimental.pallas import tpu_sc as plsc

N_GENE = 10000
N_DISEASE = 10000
E = 320000
D = 128
DW = D // 2

NC = 2
NS = 16
NW = NC * NS
PW = E // NW
G = 80
NCHUNK = PW // G


def _normalize_body(g_ref, d_ref, go_ref, do_ref):
    x = g_ref[...]
    go_ref[...] = (x / jnp.sqrt(jnp.sum(x * x, axis=1, keepdims=True))
                   ).astype(jnp.bfloat16)
    y = d_ref[...]
    do_ref[...] = (y / jnp.sqrt(jnp.sum(y * y, axis=1, keepdims=True))
                   ).astype(jnp.bfloat16)


def _normalize(h_gene, h_disease):
    return pl.pallas_call(
        _normalize_body,
        out_shape=(
            jax.ShapeDtypeStruct((N_GENE, D), jnp.bfloat16),
            jax.ShapeDtypeStruct((N_DISEASE, D), jnp.bfloat16),
        ),
    )(h_gene, h_disease)


@functools.partial(
    pl.kernel,
    mesh=plsc.VectorSubcoreMesh(core_axis_name="c", subcore_axis_name="s",
                                num_cores=NC),
    out_type=jax.ShapeDtypeStruct((E,), jnp.float32),
    compiler_params=pltpu.CompilerParams(needs_layout_passes=False,
                                         use_tc_tiling_on_sc=False),
    scratch_types=[
        pltpu.VMEM((PW,), jnp.int32),
        pltpu.VMEM((PW,), jnp.int32),
        pltpu.VMEM((G, DW), jnp.int32),
        pltpu.VMEM((G, DW), jnp.int32),
        pltpu.VMEM((G, DW), jnp.int32),
        pltpu.VMEM((G, DW), jnp.int32),
        pltpu.VMEM((G, DW), jnp.int32),
        pltpu.VMEM((G, DW), jnp.int32),
        pltpu.VMEM((G, DW), jnp.int32),
        pltpu.VMEM((G, DW), jnp.int32),
        pltpu.VMEM((PW,), jnp.float32),
        pltpu.SemaphoreType.DMA,
        pltpu.SemaphoreType.DMA,
        pltpu.SemaphoreType.DMA,
        pltpu.SemaphoreType.DMA,
    ],
)
def _edge_dots(hg_hbm, hd_hbm, src_hbm, dst_hbm, out_hbm,
               src_v, dst_v, u0, v0, u1, v1, u2, v2, u3, v3, o_v,
               s0, s1, s2, s3):
    wid = lax.axis_index("s") * NC + lax.axis_index("c")
    base = pl.multiple_of(wid * PW, 8)

    pltpu.sync_copy(src_hbm.at[pl.ds(base, PW)], src_v)
    pltpu.sync_copy(dst_hbm.at[pl.ds(base, PW)], dst_v)

    iota16 = lax.iota(jnp.int32, 16)

    def issue(c, ub, vb, sem):
        off = pl.multiple_of(c * G, 8)
        pltpu.async_copy(hg_hbm.at[src_v.at[pl.ds(off, G)]], ub, sem)
        pltpu.async_copy(hd_hbm.at[dst_v.at[pl.ds(off, G)]], vb, sem)

    def wait2(ub, vb, sem):
        pltpu.make_async_copy(hg_hbm.at[src_v.at[pl.ds(0, G)]], ub, sem).wait()
        pltpu.make_async_copy(hd_hbm.at[dst_v.at[pl.ds(0, G)]], vb, sem).wait()

    NG = G // 16

    def compute(c, ub, vb):
        off = pl.multiple_of(c * G, 8)
        zero = jnp.zeros((16,), jnp.float32)
        rows = [iota16 + (g * 16) for g in range(NG)]

        def w_body(i, accs):
            col = (iota16 + i) & (DW - 1)
            new = []
            for g in range(NG):
                uw = plsc.load_gather(ub, [rows[g], col])
                vw = plsc.load_gather(vb, [rows[g], col])
                pw = plsc.bitcast(uw, jnp.bfloat16) * \
                    plsc.bitcast(vw, jnp.bfloat16)
                p0, p1 = plsc.unpack(pw, format=plsc.PackFormat.INTERLEAVED,
                                     preferred_element_type=jnp.float32)
                new.append(accs[2 * g] + p0)
                new.append(accs[2 * g + 1] + p1)
            return tuple(new)

        accs = lax.fori_loop(0, DW, w_body, (zero,) * (2 * NG), unroll=4)
        for g in range(NG):
            o_v[pl.ds(off + g * 16, 16)] = accs[2 * g] + accs[2 * g + 1]

    bufs = ((u0, v0, s0), (u1, v1, s1), (u2, v2, s2), (u3, v3, s3))
    for b in range(4):
        issue(b, *bufs[b])

    def quad_body(t, _):
        c0 = 4 * t
        for b in range(4):
            c = c0 + b
            ub, vb, sb = bufs[b]
            wait2(ub, vb, sb)
            compute(c, ub, vb)

            @pl.when(c + 4 < NCHUNK)
            def _():
                issue(c + 4, ub, vb, sb)
        return 0

    lax.fori_loop(0, (NCHUNK - 1) // 4, quad_body, 0)
    wait2(u0, v0, s0)
    compute(NCHUNK - 1, u0, v0)
    pltpu.sync_copy(o_v, out_hbm.at[pl.ds(base, PW)])


def _pack(x_bf16):
    n = x_bf16.shape[0]
    return jax.lax.bitcast_convert_type(
        x_bf16.reshape(n, DW, 2), jnp.int32)


def kernel(h_gene, h_disease, edge_index):
    gn, dn = _normalize(h_gene, h_disease)
    src = edge_index[0].astype(jnp.int32)
    dst = edge_index[1].astype(jnp.int32)
    out = _edge_dots(_pack(gn), _pack(dn), src, dst)
    return out.reshape(E, 1)


# speedup vs baseline: 1.2398x; 1.0059x over previous
import functools

import jax
import jax.numpy as jnp
from jax import lax
from jax.experimental import pallas as pl
from jax.experimental.pallas import tpu as pltpu
from jax.experimental.pallas import tpu_sc as plsc

N_GENE = 10000
N_DISEASE = 10000
E = 320000
D = 128
DW = D // 2

NC = 2
NS = 16
NW = NC * NS
PW = E // NW
G = 128
NCHUNK = PW // G + 1


def _normalize_body(g_ref, d_ref, go_ref, do_ref):
    x = g_ref[...]
    go_ref[...] = (x / jnp.sqrt(jnp.sum(x * x, axis=1, keepdims=True))
                   ).astype(jnp.bfloat16)
    y = d_ref[...]
    do_ref[...] = (y / jnp.sqrt(jnp.sum(y * y, axis=1, keepdims=True))
                   ).astype(jnp.bfloat16)


def _normalize(h_gene, h_disease):
    return pl.pallas_call(
        _normalize_body,
        out_shape=(
            jax.ShapeDtypeStruct((N_GENE, D), jnp.bfloat16),
            jax.ShapeDtypeStruct((N_DISEASE, D), jnp.bfloat16),
        ),
    )(h_gene, h_disease)


@functools.partial(
    pl.kernel,
    mesh=plsc.VectorSubcoreMesh(core_axis_name="c", subcore_axis_name="s",
                                num_cores=NC),
    out_type=jax.ShapeDtypeStruct((E,), jnp.float32),
    compiler_params=pltpu.CompilerParams(needs_layout_passes=False,
                                         use_tc_tiling_on_sc=False),
    scratch_types=[
        pltpu.VMEM((PW,), jnp.int32),
        pltpu.VMEM((PW,), jnp.int32),
        pltpu.VMEM((G, DW), jnp.int32),
        pltpu.VMEM((G, DW), jnp.int32),
        pltpu.VMEM((G, DW), jnp.int32),
        pltpu.VMEM((G, DW), jnp.int32),
        pltpu.VMEM((G, DW), jnp.int32),
        pltpu.VMEM((G, DW), jnp.int32),
        pltpu.VMEM((G, DW), jnp.int32),
        pltpu.VMEM((G, DW), jnp.int32),
        pltpu.VMEM((PW,), jnp.float32),
        pltpu.SemaphoreType.DMA,
        pltpu.SemaphoreType.DMA,
        pltpu.SemaphoreType.DMA,
        pltpu.SemaphoreType.DMA,
    ],
)
def _edge_dots(hg_hbm, hd_hbm, src_hbm, dst_hbm, out_hbm,
               src_v, dst_v, u0, v0, u1, v1, u2, v2, u3, v3, o_v,
               s0, s1, s2, s3):
    wid = lax.axis_index("s") * NC + lax.axis_index("c")
    base = pl.multiple_of(wid * PW, 8)

    pltpu.sync_copy(src_hbm.at[pl.ds(base, PW)], src_v)
    pltpu.sync_copy(dst_hbm.at[pl.ds(base, PW)], dst_v)

    iota16 = lax.iota(jnp.int32, 16)

    def issue(c, ub, vb, sem):
        off = pl.multiple_of(jnp.minimum(c * G, PW - G), 8)
        pltpu.async_copy(hg_hbm.at[src_v.at[pl.ds(off, G)]], ub, sem)
        pltpu.async_copy(hd_hbm.at[dst_v.at[pl.ds(off, G)]], vb, sem)

    def wait2(ub, vb, sem):
        pltpu.make_async_copy(hg_hbm.at[src_v.at[pl.ds(0, G)]], ub, sem).wait()
        pltpu.make_async_copy(hd_hbm.at[dst_v.at[pl.ds(0, G)]], vb, sem).wait()

    NG = G // 16

    def compute(c, ub, vb):
        off = pl.multiple_of(jnp.minimum(c * G, PW - G), 8)
        zero = jnp.zeros((16,), jnp.float32)
        rows = [iota16 + (g * 16) for g in range(NG)]

        def w_body(i, accs):
            col = (iota16 + i) & (DW - 1)
            new = []
            for g in range(NG):
                uw = plsc.load_gather(ub, [rows[g], col])
                vw = plsc.load_gather(vb, [rows[g], col])
                pw = plsc.bitcast(uw, jnp.bfloat16) * \
                    plsc.bitcast(vw, jnp.bfloat16)
                p0, p1 = plsc.unpack(pw, format=plsc.PackFormat.INTERLEAVED,
                                     preferred_element_type=jnp.float32)
                new.append((accs[g] + p0) + p1)
            return tuple(new)

        accs = lax.fori_loop(0, DW, w_body, (zero,) * NG, unroll=2)
        for g in range(NG):
            o_v[pl.ds(off + g * 16, 16)] = accs[g]

    bufs = ((u0, v0, s0), (u1, v1, s1), (u2, v2, s2), (u3, v3, s3))
    for b in range(4):
        issue(b, *bufs[b])

    def quad_body(t, _):
        c0 = 4 * t
        for b in range(4):
            c = c0 + b
            ub, vb, sb = bufs[b]
            wait2(ub, vb, sb)
            compute(c, ub, vb)

            @pl.when(c + 4 < NCHUNK)
            def _():
                issue(c + 4, ub, vb, sb)
        return 0

    NQUAD = (NCHUNK - 3) // 4
    lax.fori_loop(0, NQUAD, quad_body, 0)
    for c in range(4 * NQUAD, NCHUNK):
        ub, vb, sb = bufs[c % 4]
        wait2(ub, vb, sb)
        compute(c, ub, vb)
    pltpu.sync_copy(o_v, out_hbm.at[pl.ds(base, PW)])


def _pack(x_bf16):
    n = x_bf16.shape[0]
    return jax.lax.bitcast_convert_type(
        x_bf16.reshape(n, DW, 2), jnp.int32)


def kernel(h_gene, h_disease, edge_index):
    gn, dn = _normalize(h_gene, h_disease)
    src = edge_index[0].astype(jnp.int32)
    dst = edge_index[1].astype(jnp.int32)
    out = _edge_dots(_pack(gn), _pack(dn), src, dst)
    return out.reshape(E, 1)


# speedup vs baseline: 1.7341x; 1.3987x over previous
import functools

import jax
import jax.numpy as jnp
from jax import lax
from jax.experimental import pallas as pl
from jax.experimental.pallas import tpu as pltpu
from jax.experimental.pallas import tpu_sc as plsc

N_GENE = 10000
N_DISEASE = 10000
E = 320000
D = 128
DW = D // 2

NC = 2
NS = 16
NW = NC * NS
PW = E // NW
G = 128
NCHUNK = PW // G + 1


def _normalize_body(g_ref, d_ref, go_ref, do_ref):
    for ref, oref in ((g_ref, go_ref), (d_ref, do_ref)):
        x = ref[...]
        xn = x / jnp.sqrt(jnp.sum(x * x, axis=1, keepdims=True))
        b = jax.lax.bitcast_convert_type(xn, jnp.uint32)
        r = (b + jnp.uint32(0x7FFF) + ((b >> 16) & jnp.uint32(1))) >> 16
        oref[...] = (r[:, :DW] | (r[:, DW:] << 16)).astype(jnp.int32)


def _normalize(h_gene, h_disease):
    return pl.pallas_call(
        _normalize_body,
        out_shape=(
            jax.ShapeDtypeStruct((N_GENE, DW), jnp.int32),
            jax.ShapeDtypeStruct((N_DISEASE, DW), jnp.int32),
        ),
    )(h_gene, h_disease)


@functools.partial(
    pl.kernel,
    mesh=plsc.VectorSubcoreMesh(core_axis_name="c", subcore_axis_name="s",
                                num_cores=NC),
    out_type=jax.ShapeDtypeStruct((E,), jnp.float32),
    compiler_params=pltpu.CompilerParams(needs_layout_passes=False,
                                         use_tc_tiling_on_sc=False),
    scratch_types=[
        pltpu.VMEM((PW,), jnp.int32),
        pltpu.VMEM((PW,), jnp.int32),
        pltpu.VMEM((G, DW), jnp.int32),
        pltpu.VMEM((G, DW), jnp.int32),
        pltpu.VMEM((G, DW), jnp.int32),
        pltpu.VMEM((G, DW), jnp.int32),
        pltpu.VMEM((G, DW), jnp.int32),
        pltpu.VMEM((G, DW), jnp.int32),
        pltpu.VMEM((G, DW), jnp.int32),
        pltpu.VMEM((G, DW), jnp.int32),
        pltpu.VMEM((PW,), jnp.float32),
        pltpu.SemaphoreType.DMA,
        pltpu.SemaphoreType.DMA,
        pltpu.SemaphoreType.DMA,
        pltpu.SemaphoreType.DMA,
    ],
)
def _edge_dots(hg_hbm, hd_hbm, src_hbm, dst_hbm, out_hbm,
               src_v, dst_v, u0, v0, u1, v1, u2, v2, u3, v3, o_v,
               s0, s1, s2, s3):
    wid = lax.axis_index("s") * NC + lax.axis_index("c")
    base = pl.multiple_of(wid * PW, 8)

    pltpu.sync_copy(src_hbm.at[pl.ds(base, PW)], src_v)
    pltpu.sync_copy(dst_hbm.at[pl.ds(base, PW)], dst_v)

    iota16 = lax.iota(jnp.int32, 16)

    def issue(c, ub, vb, sem):
        off = pl.multiple_of(jnp.minimum(c * G, PW - G), 8)
        pltpu.async_copy(hg_hbm.at[src_v.at[pl.ds(off, G)]], ub, sem)
        pltpu.async_copy(hd_hbm.at[dst_v.at[pl.ds(off, G)]], vb, sem)

    def wait2(ub, vb, sem):
        pltpu.make_async_copy(hg_hbm.at[src_v.at[pl.ds(0, G)]], ub, sem).wait()
        pltpu.make_async_copy(hd_hbm.at[dst_v.at[pl.ds(0, G)]], vb, sem).wait()

    NG = G // 16

    def compute(c, ub, vb):
        off = pl.multiple_of(jnp.minimum(c * G, PW - G), 8)
        zero = jnp.zeros((16,), jnp.float32)
        rows = [iota16 + (g * 16) for g in range(NG)]

        def w_body(i, accs):
            col = (iota16 + i) & (DW - 1)
            new = []
            for g in range(NG):
                uw = plsc.load_gather(ub, [rows[g], col])
                vw = plsc.load_gather(vb, [rows[g], col])
                pw = plsc.bitcast(uw, jnp.bfloat16) * \
                    plsc.bitcast(vw, jnp.bfloat16)
                p0, p1 = plsc.unpack(pw, format=plsc.PackFormat.INTERLEAVED,
                                     preferred_element_type=jnp.float32)
                new.append((accs[g] + p0) + p1)
            return tuple(new)

        accs = lax.fori_loop(0, DW, w_body, (zero,) * NG, unroll=2)
        for g in range(NG):
            o_v[pl.ds(off + g * 16, 16)] = accs[g]

    bufs = ((u0, v0, s0), (u1, v1, s1), (u2, v2, s2), (u3, v3, s3))
    for b in range(4):
        issue(b, *bufs[b])

    def quad_body(t, _):
        c0 = 4 * t
        for b in range(4):
            c = c0 + b
            ub, vb, sb = bufs[b]
            wait2(ub, vb, sb)
            compute(c, ub, vb)

            @pl.when(c + 4 < NCHUNK)
            def _():
                issue(c + 4, ub, vb, sb)
        return 0

    NQUAD = (NCHUNK - 3) // 4
    lax.fori_loop(0, NQUAD, quad_body, 0)
    for c in range(4 * NQUAD, NCHUNK):
        ub, vb, sb = bufs[c % 4]
        wait2(ub, vb, sb)
        compute(c, ub, vb)
    pltpu.sync_copy(o_v, out_hbm.at[pl.ds(base, PW)])


def kernel(h_gene, h_disease, edge_index):
    gi, di = _normalize(h_gene, h_disease)
    src = edge_index[0].astype(jnp.int32)
    dst = edge_index[1].astype(jnp.int32)
    out = _edge_dots(gi, di, src, dst)
    return out.reshape(E, 1)
